# Initial kernel scaffold; baseline (speedup 1.0000x reference)
import jax
import jax.numpy as jnp
from jax.experimental import pallas as pl


def kernel(x, edge_index, W1, b1, W2, b2, W3, b3, Wl, bl):
    raise NotImplementedError("write your pallas kernel here")


# speedup vs baseline: 18.1768x; 18.1768x over previous
import functools

import jax
import jax.numpy as jnp
from jax import lax
from jax.experimental import pallas as pl
from jax.experimental.pallas import tpu as pltpu
from jax.experimental.pallas import tpu_sc as plsc

N = 10000
E = 320000
NP = 10240
D = 128
H1 = 1024
H2 = 16

NC = 2
NS = 16
NW = NC * NS
RPT = NP // NW
EPT = E // NW
ECH = 80
EPS = E // NS

_mesh = plsc.VectorSubcoreMesh(
    core_axis_name="c", subcore_axis_name="s", num_cores=NC, num_subcores=NS)


def _newton_rsqrt(x16):
    i = lax.bitcast_convert_type(x16, jnp.int32)
    i = jnp.int32(0x5F3759DF) - lax.shift_right_logical(i, 1)
    y = lax.bitcast_convert_type(i, jnp.float32)
    for _ in range(3):
        y = y * (1.5 - 0.5 * x16 * y * y)
    return y


@functools.partial(
    pl.kernel,
    out_type=jax.ShapeDtypeStruct((NP,), jnp.float32),
    mesh=_mesh,
    scratch_types=[
        pltpu.VMEM((ECH,), jnp.int32),
        pltpu.VMEM((ECH,), jnp.float32),
        pltpu.VMEM((NP // NS,), jnp.float32),
        pltpu.VMEM_SHARED((NP,), jnp.float32),
    ],
)
def _sc_deg(dst_hbm, dinv_hbm, idx_v, ones_v, wk_v, deg_sh):
    c = lax.axis_index("c")
    s = lax.axis_index("s")
    wid = c * NS + s

    @pl.loop(0, (NP // NS) // 16)
    def _(i):
        wk_v[pl.ds(i * 16, 16)] = jnp.zeros((16,), jnp.float32)

    @pl.loop(0, ECH // 16)
    def _(i):
        ones_v[pl.ds(i * 16, 16)] = jnp.ones((16,), jnp.float32)

    pltpu.sync_copy(wk_v, deg_sh.at[pl.ds(s * (NP // NS), NP // NS)])
    plsc.subcore_barrier()

    base = s * EPS

    @pl.loop(0, EPS // ECH)
    def _(i):
        pltpu.sync_copy(dst_hbm.at[pl.ds(base + i * ECH, ECH)], idx_v)
        pltpu.sync_copy(ones_v, deg_sh.at[idx_v], add=True)

    plsc.subcore_barrier()

    rbase = wid * RPT
    pltpu.sync_copy(deg_sh.at[pl.ds(rbase, RPT)], wk_v.at[pl.ds(0, RPT)])

    @pl.loop(0, RPT // 16)
    def _(j):
        dvec = wk_v[pl.ds(j * 16, 16)] + 1.0
        wk_v[pl.ds(j * 16, 16)] = _newton_rsqrt(dvec)

    pltpu.sync_copy(wk_v.at[pl.ds(0, RPT)], dinv_hbm.at[pl.ds(rbase, RPT)])


@functools.partial(
    pl.kernel,
    out_type=(
        jax.ShapeDtypeStruct((NC, NP, D), jnp.float32),
        jax.ShapeDtypeStruct((NC, NP), jnp.float32),
    ),
    mesh=_mesh,
    scratch_types=[
        pltpu.VMEM((ECH,), jnp.int32),
        pltpu.VMEM((ECH,), jnp.int32),
        pltpu.VMEM((ECH, D), jnp.float32),
        pltpu.VMEM((ECH,), jnp.float32),
        pltpu.VMEM((16, D), jnp.float32),
        pltpu.VMEM_SHARED((NP, D), jnp.float32),
        pltpu.VMEM_SHARED((NP,), jnp.float32),
        pltpu.SemaphoreType.DMA,
        pltpu.SemaphoreType.DMA,
    ],
)
def _sc_agg1(src_hbm, dst_hbm, u_hbm, dinv_hbm, s2_hbm, t2_hbm,
             srcv, dstv, rows, dsc, zb, acc_s, acc_t, sem, sem2):
    c = lax.axis_index("c")
    s = lax.axis_index("s")
    wid = c * NS + s

    @pl.loop(0, 16)
    def _(r):
        for j in range(D // 16):
            zb[r, pl.ds(j * 16, 16)] = jnp.zeros((16,), jnp.float32)

    @pl.loop(0, ECH // 16)
    def _(i):
        dsc[pl.ds(i * 16, 16)] = jnp.zeros((16,), jnp.float32)

    @pl.loop(0, RPT // 16)
    def _(k):
        pltpu.sync_copy(zb, acc_s.at[pl.ds(wid * RPT + k * 16, 16), :])

    @pl.loop(0, (NP // NS) // ECH)
    def _(k):
        pltpu.sync_copy(dsc, acc_t.at[pl.ds(s * (NP // NS) + k * ECH, ECH)])

    plsc.subcore_barrier()

    ebase = wid * EPT

    @pl.loop(0, EPT // ECH)
    def _(i):
        off = ebase + i * ECH
        pltpu.sync_copy(src_hbm.at[pl.ds(off, ECH)], srcv)
        pltpu.sync_copy(dst_hbm.at[pl.ds(off, ECH)], dstv)
        pltpu.async_copy(u_hbm.at[srcv], rows, sem).wait()
        pltpu.async_copy(dinv_hbm.at[dstv], dsc, sem2).wait()
        pltpu.sync_copy(rows, acc_s.at[dstv], add=True)
        pltpu.sync_copy(dsc, acc_t.at[srcv], add=True)

    plsc.subcore_barrier()

    dbase = s * (NP // NS)
    pltpu.sync_copy(acc_s.at[pl.ds(dbase, NP // NS), :],
                    s2_hbm.at[c, pl.ds(dbase, NP // NS), :])
    pltpu.sync_copy(acc_t.at[pl.ds(dbase, NP // NS)],
                    t2_hbm.at[c, pl.ds(dbase, NP // NS)])


@functools.partial(
    pl.kernel,
    out_type=jax.ShapeDtypeStruct((NC, NP, H2), jnp.float32),
    mesh=_mesh,
    scratch_types=[
        pltpu.VMEM((ECH,), jnp.int32),
        pltpu.VMEM((ECH,), jnp.int32),
        pltpu.VMEM((ECH, H2), jnp.float32),
        pltpu.VMEM((16, H2), jnp.float32),
        pltpu.VMEM_SHARED((NP, H2), jnp.float32),
        pltpu.SemaphoreType.DMA,
    ],
    compiler_params=pltpu.CompilerParams(use_tc_tiling_on_sc=False),
)
def _sc_agg2(src_hbm, dst_hbm, v_hbm, sv_hbm, srcv, dstv, rows, zb, acc_v, sem):
    c = lax.axis_index("c")
    s = lax.axis_index("s")
    wid = c * NS + s

    @pl.loop(0, 16)
    def _(r):
        zb[r, pl.ds(0, 16)] = jnp.zeros((16,), jnp.float32)

    @pl.loop(0, RPT // 16)
    def _(k):
        pltpu.sync_copy(zb, acc_v.at[pl.ds(wid * RPT + k * 16, 16), :])

    plsc.subcore_barrier()

    ebase = wid * EPT

    @pl.loop(0, EPT // ECH)
    def _(i):
        off = ebase + i * ECH
        pltpu.sync_copy(src_hbm.at[pl.ds(off, ECH)], srcv)
        pltpu.sync_copy(dst_hbm.at[pl.ds(off, ECH)], dstv)
        pltpu.async_copy(v_hbm.at[srcv], rows, sem).wait()
        pltpu.sync_copy(rows, acc_v.at[dstv], add=True)

    plsc.subcore_barrier()

    dbase = s * (NP // NS)
    pltpu.sync_copy(acc_v.at[pl.ds(dbase, NP // NS), :],
                    sv_hbm.at[c, pl.ds(dbase, NP // NS), :])


def _tc_scale_body(dinv_ref, x_ref, u_ref):
    u_ref[...] = dinv_ref[...] * x_ref[...]


def _tc_mm_body(s2_ref, u_ref, dinv_ref, w1_ref, b1_ref, w2_ref, v_ref):
    dinv = dinv_ref[...]
    y = dinv * (s2_ref[0] + s2_ref[1] + u_ref[...])
    t1 = jnp.maximum(
        jnp.dot(y, w1_ref[...], preferred_element_type=jnp.float32) + b1_ref[...],
        0.0)
    z2 = jnp.dot(t1, w2_ref[...], preferred_element_type=jnp.float32)
    v_ref[...] = dinv * z2


def _tc_fin_body(sv_ref, v_ref, dinv_ref, tp_ref, b2_ref, w3_ref, b3_ref,
                 wl_ref, bl_ref, o_ref, acc_ref):
    i = pl.program_id(0)

    @pl.when(i == 0)
    def _():
        acc_ref[...] = jnp.zeros_like(acc_ref)

    dinv = dinv_ref[...]
    h2 = jnp.maximum(
        dinv * (sv_ref[0] + sv_ref[1] + v_ref[...]) + b2_ref[...], 0.0)
    cvec = dinv * (tp_ref[:, 0:1] + tp_ref[:, 1:2]) + dinv * dinv
    acc_ref[...] += jnp.sum(cvec * h2, axis=0, keepdims=True)

    @pl.when(i == pl.num_programs(0) - 1)
    def _():
        pooled = jnp.dot(acc_ref[...] * (1.0 / N), w3_ref[...],
                         preferred_element_type=jnp.float32) + b3_ref[...]
        o_ref[...] = jnp.dot(pooled, wl_ref[...],
                             preferred_element_type=jnp.float32) + bl_ref[...]


_RT = 400
_G = N // _RT


def kernel(x, edge_index, W1, b1, W2, b2, W3, b3, Wl, bl):
    src = edge_index[0].astype(jnp.int32)
    dst = edge_index[1].astype(jnp.int32)

    dinv_p = _sc_deg(dst)
    dinv2d = dinv_p[:N].reshape(N, 1)

    u = pl.pallas_call(
        _tc_scale_body,
        grid=(_G,),
        in_specs=[
            pl.BlockSpec((_RT, 1), lambda i: (i, 0)),
            pl.BlockSpec((_RT, D), lambda i: (i, 0)),
        ],
        out_specs=pl.BlockSpec((_RT, D), lambda i: (i, 0)),
        out_shape=jax.ShapeDtypeStruct((N, D), jnp.float32),
    )(dinv2d, x)

    s2, t2 = _sc_agg1(src, dst, u, dinv_p)

    v = pl.pallas_call(
        _tc_mm_body,
        grid=(_G,),
        in_specs=[
            pl.BlockSpec((NC, _RT, D), lambda i: (0, i, 0)),
            pl.BlockSpec((_RT, D), lambda i: (i, 0)),
            pl.BlockSpec((_RT, 1), lambda i: (i, 0)),
            pl.BlockSpec((D, H1), lambda i: (0, 0)),
            pl.BlockSpec((1, H1), lambda i: (0, 0)),
            pl.BlockSpec((H1, H2), lambda i: (0, 0)),
        ],
        out_specs=pl.BlockSpec((_RT, H2), lambda i: (i, 0)),
        out_shape=jax.ShapeDtypeStruct((N, H2), jnp.float32),
    )(s2, u, dinv2d, W1, b1.reshape(1, H1), W2)

    sv = _sc_agg2(src, dst, v)

    tp = t2.T[:N]

    out = pl.pallas_call(
        _tc_fin_body,
        grid=(_G,),
        in_specs=[
            pl.BlockSpec((NC, _RT, H2), lambda i: (0, i, 0)),
            pl.BlockSpec((_RT, H2), lambda i: (i, 0)),
            pl.BlockSpec((_RT, 1), lambda i: (i, 0)),
            pl.BlockSpec((_RT, NC), lambda i: (i, 0)),
            pl.BlockSpec((1, H2), lambda i: (0, 0)),
            pl.BlockSpec((H2, H2), lambda i: (0, 0)),
            pl.BlockSpec((1, H2), lambda i: (0, 0)),
            pl.BlockSpec((H2, 3), lambda i: (0, 0)),
            pl.BlockSpec((1, 3), lambda i: (0, 0)),
        ],
        out_specs=pl.BlockSpec((1, 3), lambda i: (0, 0)),
        out_shape=jax.ShapeDtypeStruct((1, 3), jnp.float32),
        scratch_shapes=[pltpu.VMEM((1, H2), jnp.float32)],
    )(sv, v, dinv2d, tp, b2.reshape(1, H2), W3, b3.reshape(1, H2),
      Wl, bl.reshape(1, 3))

    return out


# speedup vs baseline: 22.0511x; 1.2131x over previous
import functools

import jax
import jax.numpy as jnp
from jax import lax
from jax.experimental import pallas as pl
from jax.experimental.pallas import tpu as pltpu
from jax.experimental.pallas import tpu_sc as plsc

N = 10000
E = 320000
NP = 10240
D = 128
H1 = 1024
H2 = 16

NC = 2
NS = 16
NW = NC * NS
RPT = NP // NW
ECH = 128
EP = 327680
ER = EP // ECH
NCH = ER // NW
NB = 8
NG = NCH // NB

_mesh = plsc.VectorSubcoreMesh(
    core_axis_name="c", subcore_axis_name="s", num_cores=NC, num_subcores=NS)


def _newton_rsqrt(x16):
    i = lax.bitcast_convert_type(x16, jnp.int32)
    i = jnp.int32(0x5F3759DF) - lax.shift_right_logical(i, 1)
    y = lax.bitcast_convert_type(i, jnp.float32)
    for _ in range(3):
        y = y * (1.5 - 0.5 * x16 * y * y)
    return y


DGR = EP // 128 // NS
TR = EP // 128 // NW


@functools.partial(
    pl.kernel,
    out_type=(
        jax.ShapeDtypeStruct((NP,), jnp.float32),
        jax.ShapeDtypeStruct((NC, NP), jnp.float32),
    ),
    mesh=_mesh,
    scratch_types=[
        pltpu.VMEM((DGR, 128), jnp.int32),
        pltpu.VMEM((TR, 128), jnp.int32),
        pltpu.VMEM((TR, 128), jnp.int32),
        pltpu.VMEM((NB, 128), jnp.float32),
        pltpu.VMEM((128,), jnp.float32),
        pltpu.VMEM((NP // NS,), jnp.float32),
        pltpu.VMEM_SHARED((NP,), jnp.float32),
        pltpu.VMEM_SHARED((NP,), jnp.float32),
    ]
    + [pltpu.SemaphoreType.DMA] * (3 * NB),
)
def _sc_deg(dst128_hbm, src128_hbm, dinv_hbm, t2_hbm,
            didxd, sidxt, didxt, dsc, ones_v, wk_v, dsh, acc_t, *sems):
    dmsem = sems[:NB]
    tgsem = sems[NB:2 * NB]
    tssem = sems[2 * NB:]
    c = lax.axis_index("c")
    s = lax.axis_index("s")
    wid = c * NS + s
    sbase = s * (NP // NS)

    @pl.loop(0, (NP // NS) // 16)
    def _(i):
        wk_v[pl.ds(i * 16, 16)] = jnp.zeros((16,), jnp.float32)

    @pl.loop(0, 8)
    def _(i):
        ones_v[pl.ds(i * 16, 16)] = jnp.ones((16,), jnp.float32)

    pltpu.sync_copy(wk_v, dsh.at[pl.ds(sbase, NP // NS)])
    pltpu.sync_copy(wk_v, acc_t.at[pl.ds(sbase, NP // NS)])
    pltpu.sync_copy(dst128_hbm.at[pl.ds(s * DGR, DGR), :], didxd)
    pltpu.sync_copy(src128_hbm.at[pl.ds(wid * TR, TR), :], sidxt)
    pltpu.sync_copy(dst128_hbm.at[pl.ds(wid * TR, TR), :], didxt)
    plsc.subcore_barrier()

    @pl.loop(0, DGR // NB)
    def _(g):
        for b in range(NB):
            @pl.when(g > 0)
            def _():
                pltpu.make_async_copy(
                    ones_v, dsh.at[didxd.at[(g - 1) * NB + b]], dmsem[b]).wait()
            pltpu.async_copy(
                ones_v, dsh.at[didxd.at[g * NB + b]], dmsem[b], add=True)

    for b in range(NB):
        pltpu.make_async_copy(
            ones_v, dsh.at[didxd.at[(DGR // NB - 1) * NB + b]], dmsem[b]).wait()

    plsc.subcore_barrier()

    pltpu.sync_copy(dsh.at[pl.ds(sbase, NP // NS)], wk_v)

    @pl.loop(0, (NP // NS) // 16)
    def _(j):
        dvec = wk_v[pl.ds(j * 16, 16)] + 1.0
        wk_v[pl.ds(j * 16, 16)] = _newton_rsqrt(dvec)

    @pl.when(c == 0)
    def _():
        pltpu.sync_copy(wk_v, dinv_hbm.at[pl.ds(sbase, NP // NS)])

    pltpu.sync_copy(wk_v, dsh.at[pl.ds(sbase, NP // NS)])
    plsc.subcore_barrier()

    @pl.loop(0, TR // NB)
    def _(g):
        for b in range(NB):
            @pl.when(g > 0)
            def _():
                pltpu.make_async_copy(
                    dsc.at[b], acc_t.at[sidxt.at[(g - 1) * NB + b]],
                    tssem[b]).wait()
            pltpu.async_copy(
                dsh.at[didxt.at[g * NB + b]], dsc.at[b], tgsem[b])
        for b in range(NB):
            ci = g * NB + b
            pltpu.make_async_copy(
                dsh.at[didxt.at[ci]], dsc.at[b], tgsem[b]).wait()
            pltpu.async_copy(
                dsc.at[b], acc_t.at[sidxt.at[ci]], tssem[b], add=True)

    for b in range(NB):
        pltpu.make_async_copy(
            dsc.at[b], acc_t.at[sidxt.at[(TR // NB - 1) * NB + b]],
            tssem[b]).wait()

    plsc.subcore_barrier()

    pltpu.sync_copy(acc_t.at[pl.ds(sbase, NP // NS)],
                    t2_hbm.at[c, pl.ds(sbase, NP // NS)])


EC1 = 128
EPT = EP // NW
NC1 = EPT // EC1


@functools.partial(
    pl.kernel,
    out_type=jax.ShapeDtypeStruct((NC, NP, D), jnp.float32),
    mesh=_mesh,
    scratch_types=[
        pltpu.VMEM((EC1,), jnp.int32),
        pltpu.VMEM((EC1,), jnp.int32),
        pltpu.VMEM((EC1,), jnp.int32),
        pltpu.VMEM((EC1,), jnp.int32),
        pltpu.VMEM((2, EC1, D), jnp.float32),
        pltpu.VMEM((8, D), jnp.float32),
        pltpu.VMEM_SHARED((NP, D), jnp.float32),
    ]
    + [pltpu.SemaphoreType.DMA] * 2,
)
def _sc_agg1(srcf_hbm, dstf_hbm, u_hbm, s2_hbm,
             sidx0, sidx1, didx0, didx1, rows, zb, acc_s, gsem0, gsem1):
    sidx = (sidx0, sidx1)
    didx = (didx0, didx1)
    gsem = (gsem0, gsem1)
    c = lax.axis_index("c")
    s = lax.axis_index("s")
    wid = c * NS + s

    @pl.loop(0, 8)
    def _(r):
        for j in range(D // 16):
            zb[r, pl.ds(j * 16, 16)] = jnp.zeros((16,), jnp.float32)

    @pl.loop(0, (NP // NS) // 8)
    def _(k):
        pltpu.sync_copy(zb, acc_s.at[pl.ds(s * (NP // NS) + k * 8, 8), :])

    ebase = wid * EPT
    for b in range(2):
        pltpu.sync_copy(srcf_hbm.at[pl.ds(ebase + b * EC1, EC1)], sidx[b])
        pltpu.sync_copy(dstf_hbm.at[pl.ds(ebase + b * EC1, EC1)], didx[b])
    plsc.subcore_barrier()

    for b in range(2):
        pltpu.async_copy(u_hbm.at[sidx[b]], rows.at[b], gsem[b])

    @pl.loop(0, NC1 // 2)
    def _(h):
        for b in range(2):
            i = 2 * h + b
            pltpu.make_async_copy(
                u_hbm.at[sidx[b]], rows.at[b], gsem[b]).wait()
            pltpu.sync_copy(rows.at[b], acc_s.at[didx[b]], add=True)
            @pl.when(i + 2 < NC1)
            def _():
                off = ebase + (i + 2) * EC1
                pltpu.sync_copy(srcf_hbm.at[pl.ds(off, EC1)], sidx[b])
                pltpu.sync_copy(dstf_hbm.at[pl.ds(off, EC1)], didx[b])
                pltpu.async_copy(u_hbm.at[sidx[b]], rows.at[b], gsem[b])

    plsc.subcore_barrier()

    dbase = s * (NP // NS)
    pltpu.sync_copy(acc_s.at[pl.ds(dbase, NP // NS), :],
                    s2_hbm.at[c, pl.ds(dbase, NP // NS), :])


NB2 = 4


@functools.partial(
    pl.kernel,
    out_type=jax.ShapeDtypeStruct((NC, NP, H2), jnp.float32),
    mesh=_mesh,
    scratch_types=[
        [pltpu.VMEM((ECH,), jnp.int32) for _ in range(NB2)],
        [pltpu.VMEM((ECH,), jnp.int32) for _ in range(NB2)],
        pltpu.VMEM((NB2, ECH, H2), jnp.float32),
        pltpu.VMEM((16, H2), jnp.float32),
        pltpu.VMEM_SHARED((NP, H2), jnp.float32),
    ]
    + [pltpu.SemaphoreType.DMA] * NB2,
    compiler_params=pltpu.CompilerParams(use_tc_tiling_on_sc=False),
)
def _sc_agg2(srcf_hbm, dstf_hbm, v_hbm, sv_hbm,
             sidx, didx, rows, zb, acc_v, *gsem):
    c = lax.axis_index("c")
    s = lax.axis_index("s")
    wid = c * NS + s

    @pl.loop(0, 16)
    def _(r):
        zb[r, pl.ds(0, 16)] = jnp.zeros((16,), jnp.float32)

    @pl.loop(0, (NP // NS) // 16)
    def _(k):
        pltpu.sync_copy(zb, acc_v.at[pl.ds(s * (NP // NS) + k * 16, 16), :])

    ebase = wid * EPT
    for b in range(NB2):
        pltpu.sync_copy(srcf_hbm.at[pl.ds(ebase + b * ECH, ECH)], sidx[b])
        pltpu.sync_copy(dstf_hbm.at[pl.ds(ebase + b * ECH, ECH)], didx[b])
    plsc.subcore_barrier()

    for b in range(NB2):
        pltpu.async_copy(v_hbm.at[sidx[b]], rows.at[b], gsem[b])

    @pl.loop(0, NCH // NB2)
    def _(h):
        for b in range(NB2):
            i = NB2 * h + b
            pltpu.make_async_copy(
                v_hbm.at[sidx[b]], rows.at[b], gsem[b]).wait()
            pltpu.sync_copy(rows.at[b], acc_v.at[didx[b]], add=True)

            @pl.when(i + NB2 < NCH)
            def _():
                off = ebase + (i + NB2) * ECH
                pltpu.sync_copy(srcf_hbm.at[pl.ds(off, ECH)], sidx[b])
                pltpu.sync_copy(dstf_hbm.at[pl.ds(off, ECH)], didx[b])
                pltpu.async_copy(v_hbm.at[sidx[b]], rows.at[b], gsem[b])

    plsc.subcore_barrier()

    dbase = s * (NP // NS)
    pltpu.sync_copy(acc_v.at[pl.ds(dbase, NP // NS), :],
                    sv_hbm.at[c, pl.ds(dbase, NP // NS), :])


def _tc_scale_body(dinv_ref, x_ref, u_ref):
    u_ref[...] = dinv_ref[...] * x_ref[...]


def _tc_mm_body(s2_ref, u_ref, dinv_ref, w1_ref, b1_ref, w2_ref, v_ref):
    dinv = dinv_ref[...]
    y = dinv * (s2_ref[0] + s2_ref[1] + u_ref[...])
    t1 = jnp.maximum(
        jnp.dot(y, w1_ref[...], preferred_element_type=jnp.float32) + b1_ref[...],
        0.0)
    z2 = jnp.dot(t1, w2_ref[...], preferred_element_type=jnp.float32)
    v_ref[...] = dinv * z2


def _tc_fin_body(sv_ref, v_ref, dinv_ref, tp_ref, b2_ref, w3_ref, b3_ref,
                 wl_ref, bl_ref, o_ref, acc_ref):
    i = pl.program_id(0)

    @pl.when(i == 0)
    def _():
        acc_ref[...] = jnp.zeros_like(acc_ref)

    dinv = dinv_ref[...]
    h2 = jnp.maximum(
        dinv * (sv_ref[0] + sv_ref[1] + v_ref[...]) + b2_ref[...], 0.0)
    cvec = dinv * (tp_ref[:, 0:1] + tp_ref[:, 1:2]) + dinv * dinv
    acc_ref[...] += jnp.sum(cvec * h2, axis=0, keepdims=True)

    @pl.when(i == pl.num_programs(0) - 1)
    def _():
        pooled = jnp.dot(acc_ref[...] * (1.0 / N), w3_ref[...],
                         preferred_element_type=jnp.float32) + b3_ref[...]
        o_ref[...] = jnp.dot(pooled, wl_ref[...],
                             preferred_element_type=jnp.float32) + bl_ref[...]


_RT = 512
_G = NP // _RT
_RTF = 400
_GF = N // _RTF


def kernel(x, edge_index, W1, b1, W2, b2, W3, b3, Wl, bl):
    pad = jnp.full((EP - E,), N, dtype=jnp.int32)
    src_f = jnp.concatenate([edge_index[0].astype(jnp.int32), pad])
    dst_f = jnp.concatenate([edge_index[1].astype(jnp.int32), pad])
    src2 = src_f.reshape(ER, ECH)
    dst2 = dst_f.reshape(ER, ECH)
    x_p = jnp.pad(x, ((0, NP - N), (0, 0)))

    dinv_p, t2 = _sc_deg(dst2, src2)
    dinv2d = dinv_p.reshape(NP, 1)

    u = pl.pallas_call(
        _tc_scale_body,
        grid=(_G,),
        in_specs=[
            pl.BlockSpec((_RT, 1), lambda i: (i, 0)),
            pl.BlockSpec((_RT, D), lambda i: (i, 0)),
        ],
        out_specs=pl.BlockSpec((_RT, D), lambda i: (i, 0)),
        out_shape=jax.ShapeDtypeStruct((NP, D), jnp.float32),
    )(dinv2d, x_p)

    s2 = _sc_agg1(src_f, dst_f, u)

    v = pl.pallas_call(
        _tc_mm_body,
        grid=(_G,),
        in_specs=[
            pl.BlockSpec((NC, _RT, D), lambda i: (0, i, 0)),
            pl.BlockSpec((_RT, D), lambda i: (i, 0)),
            pl.BlockSpec((_RT, 1), lambda i: (i, 0)),
            pl.BlockSpec((D, H1), lambda i: (0, 0)),
            pl.BlockSpec((1, H1), lambda i: (0, 0)),
            pl.BlockSpec((H1, H2), lambda i: (0, 0)),
        ],
        out_specs=pl.BlockSpec((_RT, H2), lambda i: (i, 0)),
        out_shape=jax.ShapeDtypeStruct((NP, H2), jnp.float32),
    )(s2, u, dinv2d, W1, b1.reshape(1, H1), W2)

    sv = _sc_agg2(src_f, dst_f, v)

    tp = t2.T

    out = pl.pallas_call(
        _tc_fin_body,
        grid=(_GF,),
        in_specs=[
            pl.BlockSpec((NC, _RTF, H2), lambda i: (0, i, 0)),
            pl.BlockSpec((_RTF, H2), lambda i: (i, 0)),
            pl.BlockSpec((_RTF, 1), lambda i: (i, 0)),
            pl.BlockSpec((_RTF, NC), lambda i: (i, 0)),
            pl.BlockSpec((1, H2), lambda i: (0, 0)),
            pl.BlockSpec((H2, H2), lambda i: (0, 0)),
            pl.BlockSpec((1, H2), lambda i: (0, 0)),
            pl.BlockSpec((H2, 3), lambda i: (0, 0)),
            pl.BlockSpec((1, 3), lambda i: (0, 0)),
        ],
        out_specs=pl.BlockSpec((1, 3), lambda i: (0, 0)),
        out_shape=jax.ShapeDtypeStruct((1, 3), jnp.float32),
        scratch_shapes=[pltpu.VMEM((1, H2), jnp.float32)],
    )(sv, v, dinv2d, tp, b2.reshape(1, H2), W3, b3.reshape(1, H2),
      Wl, bl.reshape(1, 3))

    return out
